# baseline (speedup 1.0000x reference)
import functools

import jax
import jax.numpy as jnp
from jax import lax
from jax.experimental import pallas as pl
from jax.experimental.pallas import tpu as pltpu
from jax.experimental.pallas import tpu_sc as plsc

_K = 32
_NBUF = 3


@functools.cache
def _build(B, V, Dp):
    info = plsc.get_sparse_core_info()
    nc, ns = info.num_cores, info.num_subcores
    nw = nc * ns
    assert B % (8 * nw) == 0 and _K % 16 == 0
    b_per_w = B // nw
    assert b_per_w % _K == 0
    n_chunks = b_per_w // _K
    nt = Dp // 128

    mesh = plsc.VectorSubcoreMesh(core_axis_name="c", subcore_axis_name="s")

    def body(idx_hbm, t2_hbm, out_hbm, idx_v, *bufs):
        subs = list(bufs[:_NBUF])
        rows = list(bufs[_NBUF:2 * _NBUF])
        gs = list(bufs[2 * _NBUF:3 * _NBUF])
        ss = list(bufs[3 * _NBUF:])
        wid = lax.axis_index("s") * nc + lax.axis_index("c")
        base = wid * b_per_w
        pltpu.sync_copy(idx_hbm.at[pl.ds(base, b_per_w)], idx_v)

        def compute_sub(i, b):
            off = i * _K
            for J in range(nt):
                for kk in range(_K // 16):
                    v = idx_v[pl.ds(off + kk * 16, 16)]
                    subs[b][pl.ds(J * _K + kk * 16, 16)] = v * nt + J

        def start_gathers(b):
            for J in range(nt):
                pltpu.async_copy(
                    t2_hbm.at[subs[b].at[pl.ds(J * _K, _K)]],
                    rows[b].at[:, pl.ds(J * 128, 128)],
                    gs[b],
                )

        def wait_gathers(b):
            pltpu.make_async_copy(
                out_hbm.at[pl.ds(0, _K)], rows[b], gs[b]
            ).wait()

        def start_scatter(i, b):
            pltpu.async_copy(rows[b], out_hbm.at[pl.ds(base + i * _K, _K)], ss[b])

        def wait_scatter(b):
            pltpu.make_async_copy(
                rows[b], out_hbm.at[pl.ds(0, _K)], ss[b]
            ).wait()

        for b in range(_NBUF):
            compute_sub(b, b)
            start_gathers(b)

        @pl.loop(0, n_chunks, step=_NBUF)
        def _(g):
            for b in range(_NBUF):
                i = g + b

                @pl.when(i < n_chunks)
                def _():
                    wait_gathers(b)
                    start_scatter(i, b)

                    @pl.when(i + _NBUF < n_chunks)
                    def _():
                        compute_sub(i + _NBUF, b)
                        wait_scatter(b)
                        start_gathers(b)

        for b in range(_NBUF):
            wait_scatter(b)

    return pl.kernel(
        body,
        out_type=jax.ShapeDtypeStruct((B, Dp), jnp.float32),
        mesh=mesh,
        scratch_types=(
            [pltpu.VMEM((b_per_w,), jnp.int32)]
            + [pltpu.VMEM((nt * _K,), jnp.int32) for _ in range(_NBUF)]
            + [pltpu.VMEM((_K, Dp), jnp.float32) for _ in range(_NBUF)]
            + [pltpu.SemaphoreType.DMA for _ in range(2 * _NBUF)]
        ),
    )


def kernel(idx, table):
    Bb, L = idx.shape
    V, D = table.shape
    pad = (-D) % 128
    Dp = D + pad
    t2 = jnp.pad(table, ((0, 0), (0, pad))).reshape(V * (Dp // 128), 128)
    idx_flat = idx.reshape(-1).astype(jnp.int32)
    out = _build(Bb * L, V, Dp)(idx_flat, t2)
    return out[:, :D].reshape(Bb, L, D)


# speedup vs baseline: 1.0032x; 1.0032x over previous
import functools

import jax
import jax.numpy as jnp
from jax import lax
from jax.experimental import pallas as pl
from jax.experimental.pallas import tpu as pltpu
from jax.experimental.pallas import tpu_sc as plsc

_K = 40
_NBUF = 2
_SUBSTRIDE = 48


@functools.cache
def _build(B, V, Dp):
    info = plsc.get_sparse_core_info()
    nc, ns = info.num_cores, info.num_subcores
    nw = nc * ns
    assert B % (8 * nw) == 0 and _K % 8 == 0
    b_per_w = B // nw
    assert b_per_w % _K == 0
    n_chunks = b_per_w // _K
    nt = Dp // 128

    mesh = plsc.VectorSubcoreMesh(core_axis_name="c", subcore_axis_name="s")

    def body(idx_hbm, t2_hbm, out_hbm, idx_v, *bufs):
        subs = list(bufs[:_NBUF])
        rows = list(bufs[_NBUF:2 * _NBUF])
        gs = list(bufs[2 * _NBUF:3 * _NBUF])
        ss = list(bufs[3 * _NBUF:])
        wid = lax.axis_index("s") * nc + lax.axis_index("c")
        base = wid * b_per_w
        pltpu.sync_copy(idx_hbm.at[pl.ds(base, b_per_w)], idx_v.at[pl.ds(0, b_per_w)])

        def compute_sub(i, b):
            off = i * _K
            for J in range(nt):
                for kk in range(-(-_K // 16)):
                    v = idx_v[pl.ds(off + kk * 16, 16)]
                    subs[b][pl.ds(J * _SUBSTRIDE + kk * 16, 16)] = v * nt + J

        def start_gathers(b):
            for J in range(nt):
                pltpu.async_copy(
                    t2_hbm.at[subs[b].at[pl.ds(J * _SUBSTRIDE, _K)]],
                    rows[b].at[:, pl.ds(J * 128, 128)],
                    gs[b],
                )

        def wait_gathers(b):
            pltpu.make_async_copy(
                out_hbm.at[pl.ds(0, _K)], rows[b], gs[b]
            ).wait()

        def start_scatter(i, b):
            pltpu.async_copy(rows[b], out_hbm.at[pl.ds(base + i * _K, _K)], ss[b])

        def wait_scatter(b):
            pltpu.make_async_copy(
                rows[b], out_hbm.at[pl.ds(0, _K)], ss[b]
            ).wait()

        for b in range(_NBUF):
            compute_sub(b, b)
            start_gathers(b)

        @pl.loop(0, n_chunks, step=_NBUF)
        def _(g):
            for b in range(_NBUF):
                i = g + b

                @pl.when(i < n_chunks)
                def _():
                    wait_gathers(b)
                    start_scatter(i, b)

                    @pl.when(i + _NBUF < n_chunks)
                    def _():
                        compute_sub(i + _NBUF, b)
                        wait_scatter(b)
                        start_gathers(b)

        for b in range(_NBUF):
            wait_scatter(b)

    return pl.kernel(
        body,
        out_type=jax.ShapeDtypeStruct((B, Dp), jnp.float32),
        mesh=mesh,
        scratch_types=(
            [pltpu.VMEM((b_per_w + 16,), jnp.int32)]
            + [pltpu.VMEM((nt * _SUBSTRIDE,), jnp.int32) for _ in range(_NBUF)]
            + [pltpu.VMEM((_K, Dp), jnp.float32) for _ in range(_NBUF)]
            + [pltpu.SemaphoreType.DMA for _ in range(2 * _NBUF)]
        ),
    )


def kernel(idx, table):
    Bb, L = idx.shape
    V, D = table.shape
    pad = (-D) % 128
    Dp = D + pad
    t2 = jnp.pad(table, ((0, 0), (0, pad))).reshape(V * (Dp // 128), 128)
    idx_flat = idx.reshape(-1).astype(jnp.int32)
    out = _build(Bb * L, V, Dp)(idx_flat, t2)
    return out[:, :D].reshape(Bb, L, D)


# speedup vs baseline: 1.0049x; 1.0017x over previous
import functools

import jax
import jax.numpy as jnp
from jax import lax
from jax.experimental import pallas as pl
from jax.experimental.pallas import tpu as pltpu
from jax.experimental.pallas import tpu_sc as plsc

_K = 32
_NBUF = 2
_SUBSTRIDE = 32


@functools.cache
def _build(B, V, Dp):
    info = plsc.get_sparse_core_info()
    nc, ns = info.num_cores, info.num_subcores
    nw = nc * ns
    assert B % (8 * nw) == 0 and _K % 8 == 0
    b_per_w = B // nw
    assert b_per_w % _K == 0
    n_chunks = b_per_w // _K
    nt = Dp // 128

    mesh = plsc.VectorSubcoreMesh(core_axis_name="c", subcore_axis_name="s")

    def body(idx_hbm, t2_hbm, out_hbm, idx_v, *bufs):
        subs = list(bufs[:_NBUF])
        rows = list(bufs[_NBUF:2 * _NBUF])
        gs = list(bufs[2 * _NBUF:3 * _NBUF])
        ss = list(bufs[3 * _NBUF:])
        wid = lax.axis_index("s") * nc + lax.axis_index("c")
        base = wid * b_per_w
        pltpu.sync_copy(idx_hbm.at[pl.ds(base, b_per_w)], idx_v.at[pl.ds(0, b_per_w)])

        def compute_sub(i, b):
            off = i * _K
            for J in range(nt):
                for kk in range(-(-_K // 16)):
                    v = idx_v[pl.ds(off + kk * 16, 16)]
                    subs[b][pl.ds(J * _SUBSTRIDE + kk * 16, 16)] = v * nt + J

        def start_gathers(b):
            for J in range(nt):
                pltpu.async_copy(
                    t2_hbm.at[subs[b].at[pl.ds(J * _SUBSTRIDE, _K)]],
                    rows[b].at[:, pl.ds(J * 128, 128)],
                    gs[b],
                )

        def wait_gathers(b):
            pltpu.make_async_copy(
                out_hbm.at[pl.ds(0, _K)], rows[b], gs[b]
            ).wait()

        def start_scatter(i, b):
            pltpu.async_copy(rows[b], out_hbm.at[pl.ds(base + i * _K, _K)], ss[b])

        def wait_scatter(b):
            pltpu.make_async_copy(
                rows[b], out_hbm.at[pl.ds(0, _K)], ss[b]
            ).wait()

        for b in range(_NBUF):
            compute_sub(b, b)
            start_gathers(b)

        @pl.loop(0, n_chunks, step=_NBUF)
        def _(g):
            for b in range(_NBUF):
                i = g + b

                @pl.when(i < n_chunks)
                def _():
                    wait_gathers(b)
                    start_scatter(i, b)

                    @pl.when(i + _NBUF < n_chunks)
                    def _():
                        compute_sub(i + _NBUF, b)
                        wait_scatter(b)
                        start_gathers(b)

        for b in range(_NBUF):
            wait_scatter(b)

    return pl.kernel(
        body,
        out_type=jax.ShapeDtypeStruct((B, Dp), jnp.float32),
        mesh=mesh,
        scratch_types=(
            [pltpu.VMEM((b_per_w + 16,), jnp.int32)]
            + [pltpu.VMEM((nt * _SUBSTRIDE,), jnp.int32) for _ in range(_NBUF)]
            + [pltpu.VMEM((_K, Dp), jnp.float32) for _ in range(_NBUF)]
            + [pltpu.SemaphoreType.DMA for _ in range(2 * _NBUF)]
        ),
    )


def kernel(idx, table):
    Bb, L = idx.shape
    V, D = table.shape
    pad = (-D) % 128
    Dp = D + pad
    t2 = jnp.pad(table, ((0, 0), (0, pad))).reshape(V * (Dp // 128), 128)
    idx_flat = idx.reshape(-1).astype(jnp.int32)
    out = _build(Bb * L, V, Dp)(idx_flat, t2)
    return out[:, :D].reshape(Bb, L, D)


# speedup vs baseline: 1.0135x; 1.0085x over previous
import functools

import jax
import jax.numpy as jnp
from jax import lax
from jax.experimental import pallas as pl
from jax.experimental.pallas import tpu as pltpu
from jax.experimental.pallas import tpu_sc as plsc

_K = 32
_NBUF = 2
_SUBSTRIDE = 32


@functools.cache
def _build(B, V, Dp):
    info = plsc.get_sparse_core_info()
    nc, ns = info.num_cores, info.num_subcores
    nw = nc * ns
    assert B % (8 * nw) == 0 and _K % 8 == 0
    b_per_w = B // nw
    assert b_per_w % _K == 0
    n_chunks = b_per_w // _K
    nt = Dp // 128

    mesh = plsc.VectorSubcoreMesh(core_axis_name="c", subcore_axis_name="s")

    def body(idx_hbm, t2_hbm, out_hbm, idx_v, *bufs):
        rows = list(bufs[:_NBUF])
        gs = list(bufs[_NBUF:2 * _NBUF])
        ss = list(bufs[2 * _NBUF:])
        wid = lax.axis_index("s") * nc + lax.axis_index("c")
        base = wid * b_per_w
        pltpu.sync_copy(idx_hbm.at[pl.ds(base, b_per_w)], idx_v.at[pl.ds(0, b_per_w)])

        def start_gathers(i, b):
            pltpu.async_copy(
                t2_hbm.at[idx_v.at[pl.ds(i * _K, _K)]],
                rows[b],
                gs[b],
            )

        def wait_gathers(b):
            pltpu.make_async_copy(
                out_hbm.at[pl.ds(0, _K)], rows[b], gs[b]
            ).wait()

        def start_scatter(i, b):
            pltpu.async_copy(rows[b], out_hbm.at[pl.ds(base + i * _K, _K)], ss[b])

        def wait_scatter(b):
            pltpu.make_async_copy(
                rows[b], out_hbm.at[pl.ds(0, _K)], ss[b]
            ).wait()

        for b in range(_NBUF):
            start_gathers(b, b)

        @pl.loop(0, n_chunks, step=_NBUF)
        def _(g):
            for b in range(_NBUF):
                i = g + b

                @pl.when(i < n_chunks)
                def _():
                    wait_gathers(b)
                    start_scatter(i, b)

                    @pl.when(i + _NBUF < n_chunks)
                    def _():
                        wait_scatter(b)
                        start_gathers(i + _NBUF, b)

        for b in range(_NBUF):
            wait_scatter(b)

    return pl.kernel(
        body,
        out_type=jax.ShapeDtypeStruct((B, Dp), jnp.float32),
        mesh=mesh,
        scratch_types=(
            [pltpu.VMEM((b_per_w + 16,), jnp.int32)]
            + [pltpu.VMEM((_K, Dp), jnp.float32) for _ in range(_NBUF)]
            + [pltpu.SemaphoreType.DMA for _ in range(2 * _NBUF)]
        ),
    )


def kernel(idx, table):
    Bb, L = idx.shape
    V, D = table.shape
    pad = (-D) % 128
    Dp = D + pad
    t2 = jnp.pad(table, ((0, 0), (0, pad)))
    idx_flat = idx.reshape(-1).astype(jnp.int32)
    out = _build(Bb * L, V, Dp)(idx_flat, t2)
    return out[:, :D].reshape(Bb, L, D)
